# speedup vs baseline: 4.2843x; 4.2843x over previous
import functools

import jax
import jax.numpy as jnp
from jax import lax
from jax.experimental import pallas as pl
from jax.experimental.pallas import tpu as pltpu
from jax.experimental.pallas import tpu_sc as plsc

_V = 8192
_D = 256
_B = 64 * 1024
_NC = 2
_NS = 16
_NW = _NC * _NS
_BPW = _B // _NW
_CHUNK = 128
_NCHUNK = _BPW // _CHUNK


def _gather_body(idx_hbm, table_hbm, out_hbm, idx_v, rows_v, gsem, ssem):
    wid = lax.axis_index("s") * _NC + lax.axis_index("c")
    base = wid * _BPW

    pltpu.sync_copy(idx_hbm.at[wid], idx_v)

    def gather_start(j, b):
        pltpu.async_copy(table_hbm.at[idx_v.at[j]], rows_v.at[b], gsem)

    def gather_wait(b):
        pltpu.make_async_copy(table_hbm.at[idx_v.at[0]], rows_v.at[b], gsem).wait()

    def scatter_start(j, b):
        pltpu.async_copy(rows_v.at[b], out_hbm.at[pl.ds(base + j * _CHUNK, _CHUNK)], ssem)

    def scatter_wait(b):
        pltpu.make_async_copy(rows_v.at[b], out_hbm.at[pl.ds(base, _CHUNK)], ssem).wait()

    gather_start(0, 0)
    gather_start(1, 1)

    def step(i, carry):
        j = i * 2
        for b in range(2):
            jj = j + b
            gather_wait(b)
            scatter_start(jj, b)
            scatter_wait(b)

            @pl.when(jj + 2 < _NCHUNK)
            def _():
                gather_start(jj + 2, b)

        return carry

    lax.fori_loop(0, _NCHUNK // 2, step, 0)


_gather_call = pl.kernel(
    _gather_body,
    out_type=jax.ShapeDtypeStruct((_B, _D), jnp.float32),
    mesh=plsc.VectorSubcoreMesh(core_axis_name="c", subcore_axis_name="s"),
    scratch_types=[
        pltpu.VMEM((_NCHUNK, _CHUNK), jnp.int32),
        pltpu.VMEM((2, _CHUNK, _D), jnp.float32),
        pltpu.SemaphoreType.DMA,
        pltpu.SemaphoreType.DMA,
    ],
)


@jax.jit
def kernel(embed_id, weight):
    idx = embed_id.astype(jnp.int32).reshape(_NW, _NCHUNK, _CHUNK)
    out = _gather_call(idx, weight)
    return out.reshape(*embed_id.shape, _D)
